# Initial kernel scaffold; baseline (speedup 1.0000x reference)
import jax
import jax.numpy as jnp
from jax.experimental import pallas as pl


def kernel(x, edge_index, conv_w, conv_b, bn_w, bn_b):
    raise NotImplementedError("write your pallas kernel here")


# speedup vs baseline: 1025.8398x; 1025.8398x over previous
import functools

import jax
import jax.numpy as jnp
import numpy as np
from jax import lax
from jax.experimental import pallas as pl
from jax.experimental.pallas import tpu as pltpu
from jax.experimental.pallas import tpu_sc as plsc

_B, _C, _N, _K = 2, 128, 10000, 16
_GROUPS = 4
_NW = 32
_G = 4
_NG = 160
_RPW = _G * _NG
_R_PAD = _NW * _RPW


def _sc_gather_max(xt, idx):
    mesh = plsc.VectorSubcoreMesh(core_axis_name="c", subcore_axis_name="s")

    @functools.partial(
        pl.kernel,
        mesh=mesh,
        out_type=jax.ShapeDtypeStruct((_R_PAD, _C), jnp.float32),
        scratch_types=[
            pltpu.VMEM((_NG, 128), jnp.int32),
            pltpu.VMEM((_G * 2 * _K, _C), jnp.float32),
            pltpu.VMEM((2 * _G, _C), jnp.float32),
            pltpu.SemaphoreType.DMA,
        ],
    )
    def sc_kernel(xt_hbm, idx_hbm, y_hbm, idx_v, rows_v, out_v, sem):
        wid = lax.axis_index("s") * 2 + lax.axis_index("c")
        gbase = wid * _NG
        pltpu.sync_copy(idx_hbm.at[pl.ds(gbase, _NG)], idx_v)

        def body(jo, carry):
            for jj in range(2):
                j = jo * 2 + jj
                pltpu.async_copy(xt_hbm.at[idx_v.at[j]], rows_v, sem).wait()
                for gi in range(_G):
                    base = gi * 2 * _K
                    for ch in range(_C // 16):
                        sl = pl.ds(ch * 16, 16)
                        acc = rows_v[base, sl] - rows_v[base + _K, sl]
                        for k in range(1, _K):
                            acc = jnp.maximum(
                                acc,
                                rows_v[base + k, sl] - rows_v[base + _K + k, sl])
                        out_v[jj * _G + gi, sl] = acc
            pltpu.sync_copy(
                out_v, y_hbm.at[pl.ds((gbase + jo * 2) * _G, 2 * _G)])
            return carry

        lax.fori_loop(0, _NG // 2, body, 0)

    return sc_kernel


def _tc_body(x_ref, y_ref, a_ref, b_ref, cb_ref, bnw_ref, bnb_ref, o_ref):
    xm = x_ref[...]
    ym = y_ref[0:_B * _N, :]
    o = jnp.dot(xm, a_ref[...], preferred_element_type=jnp.float32)
    o = o + jnp.dot(ym, b_ref[...], preferred_element_type=jnp.float32)
    o = o + cb_ref[...]
    mean = jnp.mean(o, axis=0, keepdims=True)
    var = jnp.mean((o - mean) ** 2, axis=0, keepdims=True)
    o = (o - mean) * lax.rsqrt(var + 1e-5) * bnw_ref[...] + bnb_ref[...]
    o_ref[...] = 0.5 * o * (1.0 + lax.erf(o * np.float32(1.0 / np.sqrt(2.0))))


def kernel(x, edge_index, conv_w, conv_b, bn_w, bn_b):
    b, c, n, _ = x.shape
    xt = x[..., 0].transpose(0, 2, 1).reshape(b * n, c)
    ei = edge_index.astype(jnp.int32)
    off = (jnp.arange(b, dtype=jnp.int32) * n)[:, None, None]
    idx_j = ei[0] + off
    idx_i = ei[1] + off
    idx_all = jnp.concatenate(
        [idx_j.reshape(b * n, _K), idx_i.reshape(b * n, _K)], axis=-1)
    idx_pad = jnp.zeros((_R_PAD, 2 * _K), jnp.int32).at[: b * n].set(idx_all)
    idx_pad = idx_pad.reshape(_NW * _NG, _G * 2 * _K)

    y = _sc_gather_max(xt, idx_pad)(xt, idx_pad)

    w2 = conv_w[:, :, 0, 0]
    we = w2[:, 0::2]
    wo = w2[:, 1::2]
    a_m = jnp.zeros((c, c), jnp.float32)
    b_m = jnp.zeros((c, c), jnp.float32)
    for g in range(_GROUPS):
        s = 32 * g
        a_m = a_m.at[s:s + 32, s:s + 32].set(we[s:s + 32, :].T)
        b_m = b_m.at[s:s + 32, s:s + 32].set(wo[s:s + 32, :].T)

    out = pl.pallas_call(
        _tc_body,
        out_shape=jax.ShapeDtypeStruct((b * n, c), jnp.float32),
    )(xt, y, a_m, b_m,
      conv_b.reshape(1, c), bn_w.reshape(1, c), bn_b.reshape(1, c))

    return out.reshape(b, n, c).transpose(0, 2, 1)[..., None]


# speedup vs baseline: 1427.7543x; 1.3918x over previous
import functools

import jax
import jax.numpy as jnp
import numpy as np
from jax import lax
from jax.experimental import pallas as pl
from jax.experimental.pallas import tpu as pltpu
from jax.experimental.pallas import tpu_sc as plsc

_B, _C, _N, _K = 2, 128, 10000, 16
_GROUPS = 4
_NW = 32
_G = 4
_NG = 160
_RPW = _G * _NG
_R_PAD = _NW * _RPW


def _sc_gather_max(xt, idx):
    mesh = plsc.VectorSubcoreMesh(core_axis_name="c", subcore_axis_name="s")

    @functools.partial(
        pl.kernel,
        mesh=mesh,
        out_type=jax.ShapeDtypeStruct((_R_PAD, _C), jnp.float32),
        scratch_types=[
            pltpu.VMEM((_NG, 128), jnp.int32),
            pltpu.VMEM((_G * 2 * _K, _C), jnp.float32),
            pltpu.VMEM((_G * 2 * _K, _C), jnp.float32),
            pltpu.VMEM((2 * _G, _C), jnp.float32),
            pltpu.SemaphoreType.DMA,
            pltpu.SemaphoreType.DMA,
        ],
    )
    def sc_kernel(xt_hbm, idx_hbm, y_hbm, idx_v, rows_v0, rows_v1, out_v,
                  sem0, sem1):
        wid = lax.axis_index("s") * 2 + lax.axis_index("c")
        gbase = wid * _NG
        pltpu.sync_copy(idx_hbm.at[pl.ds(gbase, _NG)], idx_v)
        bufs = (rows_v0, rows_v1)
        sems = (sem0, sem1)

        pltpu.async_copy(xt_hbm.at[idx_v.at[0]], rows_v0, sem0)
        pltpu.async_copy(xt_hbm.at[idx_v.at[1]], rows_v1, sem1)

        def body(jo, carry):
            for jj in range(2):
                j = jo * 2 + jj
                rows_v = bufs[jj]
                pltpu.make_async_copy(
                    xt_hbm.at[idx_v.at[j]], rows_v, sems[jj]).wait()
                for gi in range(_G):
                    base = gi * 2 * _K
                    for ch in range(_C // 16):
                        sl = pl.ds(ch * 16, 16)
                        acc = rows_v[base, sl] - rows_v[base + _K, sl]
                        for k in range(1, _K):
                            acc = jnp.maximum(
                                acc,
                                rows_v[base + k, sl] - rows_v[base + _K + k, sl])
                        out_v[jj * _G + gi, sl] = acc

                @pl.when(j + 2 < _NG)
                def _():
                    pltpu.async_copy(
                        xt_hbm.at[idx_v.at[j + 2]], bufs[jj], sems[jj])

            pltpu.sync_copy(
                out_v, y_hbm.at[pl.ds((gbase + jo * 2) * _G, 2 * _G)])
            return carry

        lax.fori_loop(0, _NG // 2, body, 0)

    return sc_kernel


def _tc_body(x_ref, y_ref, a_ref, b_ref, cb_ref, bnw_ref, bnb_ref, o_ref):
    xm = x_ref[...]
    ym = y_ref[0:_B * _N, :]
    o = jnp.dot(xm, a_ref[...], preferred_element_type=jnp.float32)
    o = o + jnp.dot(ym, b_ref[...], preferred_element_type=jnp.float32)
    o = o + cb_ref[...]
    mean = jnp.mean(o, axis=0, keepdims=True)
    var = jnp.mean((o - mean) ** 2, axis=0, keepdims=True)
    o = (o - mean) * lax.rsqrt(var + 1e-5) * bnw_ref[...] + bnb_ref[...]
    o_ref[...] = 0.5 * o * (1.0 + lax.erf(o * np.float32(1.0 / np.sqrt(2.0))))


def kernel(x, edge_index, conv_w, conv_b, bn_w, bn_b):
    b, c, n, _ = x.shape
    xt = x[..., 0].transpose(0, 2, 1).reshape(b * n, c)
    ei = edge_index.astype(jnp.int32)
    off = (jnp.arange(b, dtype=jnp.int32) * n)[:, None, None]
    idx_j = ei[0] + off
    idx_i = ei[1] + off
    idx_all = jnp.concatenate(
        [idx_j.reshape(b * n, _K), idx_i.reshape(b * n, _K)], axis=-1)
    idx_pad = jnp.zeros((_R_PAD, 2 * _K), jnp.int32).at[: b * n].set(idx_all)
    idx_pad = idx_pad.reshape(_NW * _NG, _G * 2 * _K)

    y = _sc_gather_max(xt, idx_pad)(xt, idx_pad)

    w2 = conv_w[:, :, 0, 0]
    we = w2[:, 0::2]
    wo = w2[:, 1::2]
    a_m = jnp.zeros((c, c), jnp.float32)
    b_m = jnp.zeros((c, c), jnp.float32)
    for g in range(_GROUPS):
        s = 32 * g
        a_m = a_m.at[s:s + 32, s:s + 32].set(we[s:s + 32, :].T)
        b_m = b_m.at[s:s + 32, s:s + 32].set(wo[s:s + 32, :].T)

    out = pl.pallas_call(
        _tc_body,
        out_shape=jax.ShapeDtypeStruct((b * n, c), jnp.float32),
    )(xt, y, a_m, b_m,
      conv_b.reshape(1, c), bn_w.reshape(1, c), bn_b.reshape(1, c))

    return out.reshape(b, n, c).transpose(0, 2, 1)[..., None]
